# Initial kernel scaffold; baseline (speedup 1.0000x reference)
import functools

import jax
import jax.numpy as jnp
from jax import lax
from jax.experimental import pallas as pl
from jax.experimental.pallas import tpu as pltpu
from jax.experimental.pallas import tpu_sc as plsc

N = 10000
D = 128
NC = 2
NS = 16
L = 16
NW = NC * NS
K = 128
NPAD = 10240
RPT = NPAD // NS

_mesh = plsc.VectorSubcoreMesh(core_axis_name="c", subcore_axis_name="s")


def _hist_call(cols2d, cpt):
    @functools.partial(
        pl.kernel,
        mesh=_mesh,
        out_type=jax.ShapeDtypeStruct((NC, NPAD, L), jnp.float32),
        scratch_types=[
            pltpu.VMEM((cpt, K), jnp.int32),
            pltpu.VMEM((K, L), jnp.float32),
            pltpu.VMEM((RPT, L), jnp.float32),
            pltpu.VMEM_SHARED((NPAD, L), jnp.float32),
            pltpu.SemaphoreType.DMA,
        ],
    )
    def hist(cols_hbm, deg_hbm, colv, onehot, zbuf, deg_sh, sem):
        cid = lax.axis_index("c")
        sid = lax.axis_index("s")
        wid = cid * NS + sid
        lane = lax.iota(jnp.int32, L)
        one_row = jnp.where(lane == 0, 1.0, 0.0).astype(jnp.float32)
        zero_row = jnp.zeros((L,), jnp.float32)

        @pl.loop(0, RPT)
        def _(i):
            zbuf[i, :] = zero_row

        @pl.loop(0, K)
        def _(i):
            onehot[i, :] = one_row

        pltpu.sync_copy(zbuf, deg_sh.at[pl.ds(sid * RPT, RPT)])
        plsc.subcore_barrier()

        pltpu.async_copy(cols_hbm.at[pl.ds(wid * cpt, cpt)], colv, sem).wait()

        @pl.loop(0, cpt)
        def _(c):
            pltpu.sync_copy(onehot, deg_sh.at[colv.at[c]], add=True)

        plsc.subcore_barrier()
        pltpu.sync_copy(
            deg_sh.at[pl.ds(sid * RPT, RPT)],
            deg_hbm.at[cid].at[pl.ds(sid * RPT, RPT)],
        )

    return hist(cols2d)


def _spmm_call(rows2d, cols2d, y, cpt):
    @functools.partial(
        pl.kernel,
        mesh=_mesh,
        out_type=jax.ShapeDtypeStruct((NC, NPAD, D), jnp.float32),
        scratch_types=[
            pltpu.VMEM((cpt, K), jnp.int32),
            pltpu.VMEM((cpt, K), jnp.int32),
            pltpu.VMEM((2, K, D), jnp.float32),
            pltpu.VMEM_SHARED((NPAD, D), jnp.float32),
            pltpu.SemaphoreType.DMA,
            pltpu.SemaphoreType.DMA,
            pltpu.SemaphoreType.DMA,
        ],
    )
    def spmm(rows_hbm, cols_hbm, y_hbm, acc_hbm, rowv, colv, gbuf, acc_sh,
             sem0, sem1, semi):
        cid = lax.axis_index("c")
        sid = lax.axis_index("s")
        wid = cid * NS + sid
        zero_row = jnp.zeros((L,), jnp.float32)

        @pl.loop(0, K)
        def _(i):
            @pl.loop(0, D // L)
            def _(j):
                gbuf[0, i, pl.ds(j * L, L)] = zero_row

        @pl.loop(0, RPT // K)
        def _(i):
            pltpu.sync_copy(gbuf.at[0], acc_sh.at[pl.ds(sid * RPT + i * K, K)])

        plsc.subcore_barrier()

        pltpu.async_copy(rows_hbm.at[pl.ds(wid * cpt, cpt)], rowv, semi).wait()
        pltpu.async_copy(cols_hbm.at[pl.ds(wid * cpt, cpt)], colv, semi).wait()

        sems = (sem0, sem1)

        def gather(c, b):
            return pltpu.async_copy(y_hbm.at[colv.at[c]], gbuf.at[b], sems[b])

        pend = [gather(0, 0)]
        if cpt > 1:
            pend.append(gather(1, 1))
        for c in range(cpt):
            b = c & 1
            pend[b].wait()
            pltpu.sync_copy(gbuf.at[b], acc_sh.at[rowv.at[c]], add=True)
            if c + 2 < cpt:
                pend[b] = gather(c + 2, b)

        plsc.subcore_barrier()
        pltpu.sync_copy(
            acc_sh.at[pl.ds(sid * RPT, RPT)],
            acc_hbm.at[cid].at[pl.ds(sid * RPT, RPT)],
        )

    return spmm(rows2d, cols2d, y)


def _scale_matmul_body(x_ref, wt_ref, d0_ref, d1_ref, y_ref):
    deg = d0_ref[0][:, :1] + d1_ref[0][:, :1] + 1.0
    xhat = jnp.dot(
        x_ref[...], wt_ref[...],
        preferred_element_type=jnp.float32,
        precision=lax.Precision.HIGHEST,
    )
    y_ref[...] = lax.rsqrt(deg) * xhat


def _scale_matmul_call(x_pad, wt, degs):
    bm = 512
    return pl.pallas_call(
        _scale_matmul_body,
        grid=(NPAD // bm,),
        in_specs=[
            pl.BlockSpec((bm, D), lambda i: (i, 0)),
            pl.BlockSpec((D, D), lambda i: (0, 0)),
            pl.BlockSpec((1, bm, L), lambda i: (0, i, 0)),
            pl.BlockSpec((1, bm, L), lambda i: (1, i, 0)),
        ],
        out_specs=pl.BlockSpec((bm, D), lambda i: (i, 0)),
        out_shape=jax.ShapeDtypeStruct((NPAD, D), jnp.float32),
    )(x_pad, wt, degs, degs)


def _finalize_body(a0_ref, a1_ref, y_ref, d0_ref, d1_ref, o_ref):
    deg = d0_ref[0][:, :1] + d1_ref[0][:, :1] + 1.0
    s = a0_ref[0] + a1_ref[0] + y_ref[...]
    o_ref[...] = lax.rsqrt(deg) * s


def _finalize_call(accs, y, degs):
    bm = 1000
    return pl.pallas_call(
        _finalize_body,
        grid=(N // bm,),
        in_specs=[
            pl.BlockSpec((1, bm, D), lambda i: (0, i, 0)),
            pl.BlockSpec((1, bm, D), lambda i: (1, i, 0)),
            pl.BlockSpec((bm, D), lambda i: (i, 0)),
            pl.BlockSpec((1, bm, L), lambda i: (0, i, 0)),
            pl.BlockSpec((1, bm, L), lambda i: (1, i, 0)),
        ],
        out_specs=pl.BlockSpec((bm, D), lambda i: (i, 0)),
        out_shape=jax.ShapeDtypeStruct((N, D), jnp.float32),
    )(accs, accs, y, degs, degs)


def kernel(edge_index, X, W):
    E = edge_index.shape[1]
    rows = edge_index[0].astype(jnp.int32)
    cols = edge_index[1].astype(jnp.int32)

    cpt = -(-E // (K * NW))
    e_pad = K * NW * cpt
    npad_e = e_pad - E
    pad_idx = N + jnp.arange(npad_e, dtype=jnp.int32) % (NPAD - N)
    rows2d = jnp.concatenate([rows, pad_idx]).reshape(e_pad // K, K)
    cols2d = jnp.concatenate([cols, pad_idx]).reshape(e_pad // K, K)

    x_pad = jnp.pad(X, ((0, NPAD - N), (0, 0)))
    wt = W.T

    degs = _hist_call(cols2d, cpt)
    y = _scale_matmul_call(x_pad, wt, degs)
    accs = _spmm_call(rows2d, cols2d, y, cpt)
    return _finalize_call(accs, y, degs)


# speedup vs baseline: 36.0772x; 36.0772x over previous
import dataclasses
import functools

import jax
import jax.numpy as jnp
from jax import lax
from jax.experimental import pallas as pl
from jax.experimental.pallas import tpu as pltpu
from jax.experimental.pallas import tpu_sc as plsc

N = 10000
D = 128
NC = 2
NS = 16
L = 16
NW = NC * NS
K = 128
NPAD = 10240
RPT = NPAD // NS

_mesh = plsc.VectorSubcoreMesh(core_axis_name="c", subcore_axis_name="s")

_sc_params = pltpu.CompilerParams()
if "needs_layout_passes" in pltpu.CompilerParams.__dataclass_fields__:
    _sc_params = dataclasses.replace(_sc_params, needs_layout_passes=False)


def _hist_call(idx3d, cpt):
    @functools.partial(
        pl.kernel,
        mesh=_mesh,
        out_type=jax.ShapeDtypeStruct((NC, NPAD, D), jnp.float32),
        compiler_params=_sc_params,
        scratch_types=[
            pltpu.VMEM((cpt, 2, K), jnp.int32),
            pltpu.VMEM((NPAD,), jnp.float32),
            pltpu.VMEM((NS, RPT), jnp.float32),
            pltpu.VMEM((L, D), jnp.float32),
            pltpu.VMEM_SHARED((NS, NPAD), jnp.float32),
            pltpu.SemaphoreType.DMA,
        ],
    )
    def hist(idx_hbm, deg_hbm, idxv, histv, redv, bcast, stage_sh, sem):
        cid = lax.axis_index("c")
        sid = lax.axis_index("s")
        wid = cid * NS + sid
        zero = jnp.zeros((L,), jnp.float32)

        @pl.loop(0, NPAD // L)
        def _(i):
            histv[pl.ds(i * L, L)] = zero

        pltpu.async_copy(idx_hbm.at[pl.ds(wid * cpt, cpt)], idxv, sem).wait()

        lane = lax.iota(jnp.int32, L)
        pos = lane.astype(jnp.float32)
        last_lane = lane == (L - 1)
        first_lane = lane == 0
        nxt = jnp.minimum(lane + 1, L - 1)
        prv = jnp.maximum(lane - 1, 0)

        @pl.loop(0, cpt)
        def _(c):
            @pl.loop(0, K // L)
            def _(j):
                s = lax.sort(idxv[c, 0, pl.ds(j * L, L)])
                s_next = s.at[nxt].get(mode="promise_in_bounds")
                s_prev = s.at[prv].get(mode="promise_in_bounds")
                is_last = (s != s_next) | last_lane
                is_first = (s != s_prev) | first_lane
                plsc.addupdate_scatter(histv, [s], -pos, mask=is_first)
                plsc.addupdate_scatter(histv, [s], pos + 1.0, mask=is_last)

        pltpu.sync_copy(histv, stage_sh.at[sid])
        plsc.subcore_barrier()

        pltpu.sync_copy(stage_sh.at[:, pl.ds(sid * RPT, RPT)], redv)

        @pl.loop(0, RPT // L)
        def _(i):
            v = redv[0, pl.ds(i * L, L)]
            for r in range(1, NS):
                v = v + redv[r, pl.ds(i * L, L)]
            histv[pl.ds(i * L, L)] = v

        @pl.loop(0, RPT // L)
        def _(g):
            v = histv[pl.ds(g * L, L)]
            for r in range(L):
                row = zero + v[r]
                for j in range(D // L):
                    bcast[r, pl.ds(j * L, L)] = row
            pltpu.sync_copy(
                bcast, deg_hbm.at[cid, pl.ds(sid * RPT + g * L, L)])

    return hist(idx3d)


def _spmm_call(idx3d, y, cpt):
    @functools.partial(
        pl.kernel,
        mesh=_mesh,
        out_type=jax.ShapeDtypeStruct((NC, NPAD, D), jnp.float32),
        scratch_types=[
            pltpu.VMEM((2, 2, K), jnp.int32),
            pltpu.VMEM((2, K, D), jnp.float32),
            pltpu.VMEM_SHARED((NPAD, D), jnp.float32),
            pltpu.SemaphoreType.DMA,
            pltpu.SemaphoreType.DMA,
            pltpu.SemaphoreType.DMA,
            pltpu.SemaphoreType.DMA,
        ],
    )
    def spmm(idx_hbm, y_hbm, acc_hbm, idxv, gbuf, acc_sh, gs0, gs1, is0, is1):
        cid = lax.axis_index("c")
        sid = lax.axis_index("s")
        wid = cid * NS + sid
        base = wid * cpt
        zero_row = jnp.zeros((L,), jnp.float32)

        @pl.loop(0, K)
        def _(i):
            @pl.loop(0, D // L)
            def _(j):
                gbuf[0, i, pl.ds(j * L, L)] = zero_row

        @pl.loop(0, RPT // K)
        def _(i):
            pltpu.sync_copy(gbuf.at[0], acc_sh.at[pl.ds(sid * RPT + i * K, K)])

        plsc.subcore_barrier()

        gsems = (gs0, gs1)
        isems = (is0, is1)

        def load_idx(c, b):
            return pltpu.async_copy(idx_hbm.at[base + c], idxv.at[b], isems[b])

        def gather(b):
            return pltpu.async_copy(
                y_hbm.at[idxv.at[b, 0]], gbuf.at[b], gsems[b])

        def scatter(b):
            pltpu.sync_copy(gbuf.at[b], acc_sh.at[idxv.at[b, 1]], add=True)

        for b in range(2):
            load_idx(b, b).wait()
        gpend = [gather(0), gather(1)]

        @pl.loop(0, cpt // 2 - 1)
        def _(p):
            c = p * 2
            for b in range(2):
                gpend[b].wait()
                scatter(b)
                load_idx(c + 2 + b, b).wait()
                gpend[b] = gather(b)

        for b in range(2):
            gpend[b].wait()
            scatter(b)

        plsc.subcore_barrier()
        pltpu.sync_copy(
            acc_sh.at[pl.ds(sid * RPT, RPT)],
            acc_hbm.at[cid].at[pl.ds(sid * RPT, RPT)],
        )

    return spmm(idx3d, y)


def _scale_matmul_body(x_ref, wt_ref, d0_ref, d1_ref, y_ref):
    deg = d0_ref[0] + d1_ref[0] + 1.0
    xhat = jnp.dot(
        x_ref[...], wt_ref[...],
        preferred_element_type=jnp.float32,
        precision=lax.Precision.HIGHEST,
    )
    y_ref[...] = lax.rsqrt(deg) * xhat


def _scale_matmul_call(x_pad, wt, degs):
    bm = 512
    return pl.pallas_call(
        _scale_matmul_body,
        grid=(NPAD // bm,),
        in_specs=[
            pl.BlockSpec((bm, D), lambda i: (i, 0)),
            pl.BlockSpec((D, D), lambda i: (0, 0)),
            pl.BlockSpec((1, bm, D), lambda i: (0, i, 0)),
            pl.BlockSpec((1, bm, D), lambda i: (1, i, 0)),
        ],
        out_specs=pl.BlockSpec((bm, D), lambda i: (i, 0)),
        out_shape=jax.ShapeDtypeStruct((NPAD, D), jnp.float32),
    )(x_pad, wt, degs, degs)


def _finalize_body(a0_ref, a1_ref, y_ref, d0_ref, d1_ref, o_ref):
    deg = d0_ref[0] + d1_ref[0] + 1.0
    s = a0_ref[0] + a1_ref[0] + y_ref[...]
    o_ref[...] = lax.rsqrt(deg) * s


def _finalize_call(accs, y, degs):
    bm = 1000
    return pl.pallas_call(
        _finalize_body,
        grid=(N // bm,),
        in_specs=[
            pl.BlockSpec((1, bm, D), lambda i: (0, i, 0)),
            pl.BlockSpec((1, bm, D), lambda i: (1, i, 0)),
            pl.BlockSpec((bm, D), lambda i: (i, 0)),
            pl.BlockSpec((1, bm, D), lambda i: (0, i, 0)),
            pl.BlockSpec((1, bm, D), lambda i: (1, i, 0)),
        ],
        out_specs=pl.BlockSpec((bm, D), lambda i: (i, 0)),
        out_shape=jax.ShapeDtypeStruct((N, D), jnp.float32),
    )(accs, accs, y, degs, degs)


def kernel(edge_index, X, W):
    E = edge_index.shape[1]
    rows = edge_index[0].astype(jnp.int32)
    cols = edge_index[1].astype(jnp.int32)

    cpt = -(-E // (K * NW))
    cpt = (cpt + 7) // 8 * 8
    e_pad = K * NW * cpt
    npad_e = e_pad - E
    pad_idx = N + jnp.arange(npad_e, dtype=jnp.int32) % (NPAD - N)
    rows2d = jnp.concatenate([rows, pad_idx]).reshape(e_pad // K, K)
    cols2d = jnp.concatenate([cols, pad_idx]).reshape(e_pad // K, K)
    idx3d = jnp.stack([cols2d, rows2d], axis=1)

    x_pad = jnp.pad(X, ((0, NPAD - N), (0, 0)))
    wt = W.T

    degs = _hist_call(idx3d, cpt)
    y = _scale_matmul_call(x_pad, wt, degs)
    accs = _spmm_call(idx3d, y, cpt)
    return _finalize_call(accs, y, degs)


# speedup vs baseline: 39.1525x; 1.0852x over previous
import dataclasses
import functools

import jax
import jax.numpy as jnp
from jax import lax
from jax.experimental import pallas as pl
from jax.experimental.pallas import tpu as pltpu
from jax.experimental.pallas import tpu_sc as plsc

N = 10000
D = 128
NC = 2
NS = 16
L = 16
NW = NC * NS
K = 128
NPAD = 10240
RPT = NPAD // NS

_mesh = plsc.VectorSubcoreMesh(core_axis_name="c", subcore_axis_name="s")

_sc_params = pltpu.CompilerParams()
if "needs_layout_passes" in pltpu.CompilerParams.__dataclass_fields__:
    _sc_params = dataclasses.replace(_sc_params, needs_layout_passes=False)


def _hist_call(idx3d, cpt):
    @functools.partial(
        pl.kernel,
        mesh=_mesh,
        out_type=jax.ShapeDtypeStruct((NC, NPAD, D), jnp.float32),
        compiler_params=_sc_params,
        scratch_types=[
            pltpu.VMEM((cpt, 2, K), jnp.int32),
            pltpu.VMEM((NPAD,), jnp.float32),
            pltpu.VMEM((NS, RPT), jnp.float32),
            pltpu.VMEM((L, D), jnp.float32),
            pltpu.VMEM_SHARED((NS, NPAD), jnp.float32),
            pltpu.SemaphoreType.DMA,
        ],
    )
    def hist(idx_hbm, deg_hbm, idxv, histv, redv, bcast, stage_sh, sem):
        cid = lax.axis_index("c")
        sid = lax.axis_index("s")
        wid = cid * NS + sid
        zero = jnp.zeros((L,), jnp.float32)

        @pl.loop(0, NPAD // L)
        def _(i):
            histv[pl.ds(i * L, L)] = zero

        pltpu.async_copy(idx_hbm.at[pl.ds(wid * cpt, cpt)], idxv, sem).wait()

        lane = lax.iota(jnp.int32, L)
        pos = lane.astype(jnp.float32)
        last_lane = lane == (L - 1)
        first_lane = lane == 0
        nxt = jnp.minimum(lane + 1, L - 1)
        prv = jnp.maximum(lane - 1, 0)

        @pl.loop(0, cpt)
        def _(c):
            @pl.loop(0, K // L)
            def _(j):
                s = lax.sort(idxv[c, 0, pl.ds(j * L, L)])
                s_next = s.at[nxt].get(mode="promise_in_bounds")
                s_prev = s.at[prv].get(mode="promise_in_bounds")
                is_last = (s != s_next) | last_lane
                is_first = (s != s_prev) | first_lane
                plsc.addupdate_scatter(histv, [s], -pos, mask=is_first)
                plsc.addupdate_scatter(histv, [s], pos + 1.0, mask=is_last)

        pltpu.sync_copy(histv, stage_sh.at[sid])
        plsc.subcore_barrier()

        pltpu.sync_copy(stage_sh.at[:, pl.ds(sid * RPT, RPT)], redv)

        @pl.loop(0, RPT // L)
        def _(i):
            v = redv[0, pl.ds(i * L, L)]
            for r in range(1, NS):
                v = v + redv[r, pl.ds(i * L, L)]
            histv[pl.ds(i * L, L)] = v

        @pl.loop(0, RPT // L)
        def _(g):
            v = histv[pl.ds(g * L, L)]
            for r in range(L):
                row = zero + v[r]
                for j in range(D // L):
                    bcast[r, pl.ds(j * L, L)] = row
            pltpu.sync_copy(
                bcast, deg_hbm.at[cid, pl.ds(sid * RPT + g * L, L)])

    return hist(idx3d)


def _spmm_call(idx3d, y, cpt):
    @functools.partial(
        pl.kernel,
        mesh=_mesh,
        out_type=jax.ShapeDtypeStruct((NC, NPAD, D), jnp.float32),
        compiler_params=_sc_params,
        scratch_types=[
            pltpu.VMEM((4, 2, K), jnp.int32),
            pltpu.VMEM((2, K, D), jnp.float32),
            pltpu.VMEM_SHARED((NPAD, D), jnp.float32),
            pltpu.SemaphoreType.DMA,
            pltpu.SemaphoreType.DMA,
            pltpu.SemaphoreType.DMA,
            pltpu.SemaphoreType.DMA,
            pltpu.SemaphoreType.DMA,
            pltpu.SemaphoreType.DMA,
        ],
    )
    def spmm(idx_hbm, y_hbm, acc_hbm, idxv, gbuf, acc_sh,
             gs0, gs1, is0, is1, is2, is3):
        cid = lax.axis_index("c")
        sid = lax.axis_index("s")
        wid = cid * NS + sid
        base = wid * cpt
        zero_row = jnp.zeros((L,), jnp.float32)

        @pl.loop(0, K)
        def _(i):
            @pl.loop(0, D // L)
            def _(j):
                gbuf[0, i, pl.ds(j * L, L)] = zero_row

        @pl.loop(0, RPT // K)
        def _(i):
            pltpu.sync_copy(gbuf.at[0], acc_sh.at[pl.ds(sid * RPT + i * K, K)])

        plsc.subcore_barrier()

        gsems = (gs0, gs1)
        isems = (is0, is1, is2, is3)

        def load_idx(c, q):
            return pltpu.async_copy(idx_hbm.at[base + c], idxv.at[q], isems[q])

        def idx_wait(q):
            pltpu.make_async_copy(
                idx_hbm.at[base], idxv.at[q], isems[q]).wait()

        def gather(q, b):
            return pltpu.async_copy(
                y_hbm.at[idxv.at[q, 0]], gbuf.at[b], gsems[b])

        def scatter(q, b):
            pltpu.sync_copy(gbuf.at[b], acc_sh.at[idxv.at[q, 1]], add=True)

        for q in range(4):
            load_idx(q, q)
        gpend = [None, None]
        for b in range(2):
            idx_wait(b)
            gpend[b] = gather(b, b)

        @pl.loop(0, cpt // 4 - 1)
        def _(p):
            c = p * 4
            for k in range(4):
                b = k & 1
                q2 = (k + 2) & 3
                gpend[b].wait()
                scatter(k, b)
                load_idx(c + k + 4, k)
                idx_wait(q2)
                gpend[b] = gather(q2, b)

        for k in range(4):
            b = k & 1
            q2 = (k + 2) & 3
            gpend[b].wait()
            scatter(k, b)
            if k < 2:
                idx_wait(q2)
                gpend[b] = gather(q2, b)

        plsc.subcore_barrier()
        pltpu.sync_copy(
            acc_sh.at[pl.ds(sid * RPT, RPT)],
            acc_hbm.at[cid].at[pl.ds(sid * RPT, RPT)],
        )

    return spmm(idx3d, y)


def _scale_matmul_body(x_ref, wt_ref, d0_ref, d1_ref, y_ref):
    deg = d0_ref[0] + d1_ref[0] + 1.0
    xhat = jnp.dot(
        x_ref[...], wt_ref[...],
        preferred_element_type=jnp.float32,
        precision=lax.Precision.HIGHEST,
    )
    y_ref[...] = lax.rsqrt(deg) * xhat


def _scale_matmul_call(x_pad, wt, degs):
    bm = 512
    return pl.pallas_call(
        _scale_matmul_body,
        grid=(NPAD // bm,),
        in_specs=[
            pl.BlockSpec((bm, D), lambda i: (i, 0)),
            pl.BlockSpec((D, D), lambda i: (0, 0)),
            pl.BlockSpec((1, bm, D), lambda i: (0, i, 0)),
            pl.BlockSpec((1, bm, D), lambda i: (1, i, 0)),
        ],
        out_specs=pl.BlockSpec((bm, D), lambda i: (i, 0)),
        out_shape=jax.ShapeDtypeStruct((NPAD, D), jnp.float32),
    )(x_pad, wt, degs, degs)


def _finalize_body(a0_ref, a1_ref, y_ref, d0_ref, d1_ref, o_ref):
    deg = d0_ref[0] + d1_ref[0] + 1.0
    s = a0_ref[0] + a1_ref[0] + y_ref[...]
    o_ref[...] = lax.rsqrt(deg) * s


def _finalize_call(accs, y, degs):
    bm = 1000
    return pl.pallas_call(
        _finalize_body,
        grid=(N // bm,),
        in_specs=[
            pl.BlockSpec((1, bm, D), lambda i: (0, i, 0)),
            pl.BlockSpec((1, bm, D), lambda i: (1, i, 0)),
            pl.BlockSpec((bm, D), lambda i: (i, 0)),
            pl.BlockSpec((1, bm, D), lambda i: (0, i, 0)),
            pl.BlockSpec((1, bm, D), lambda i: (1, i, 0)),
        ],
        out_specs=pl.BlockSpec((bm, D), lambda i: (i, 0)),
        out_shape=jax.ShapeDtypeStruct((N, D), jnp.float32),
    )(accs, accs, y, degs, degs)


def kernel(edge_index, X, W):
    E = edge_index.shape[1]
    rows = edge_index[0].astype(jnp.int32)
    cols = edge_index[1].astype(jnp.int32)

    cpt = -(-E // (K * NW))
    cpt = (cpt + 7) // 8 * 8
    e_pad = K * NW * cpt
    npad_e = e_pad - E
    pad_idx = N + jnp.arange(npad_e, dtype=jnp.int32) % (NPAD - N)
    rows2d = jnp.concatenate([rows, pad_idx]).reshape(e_pad // K, K)
    cols2d = jnp.concatenate([cols, pad_idx]).reshape(e_pad // K, K)
    idx3d = jnp.stack([cols2d, rows2d], axis=1)

    x_pad = jnp.pad(X, ((0, NPAD - N), (0, 0)))
    wt = W.T

    degs = _hist_call(idx3d, cpt)
    y = _scale_matmul_call(x_pad, wt, degs)
    accs = _spmm_call(idx3d, y, cpt)
    return _finalize_call(accs, y, degs)


# speedup vs baseline: 40.7931x; 1.0419x over previous
import dataclasses
import functools

import jax
import jax.numpy as jnp
from jax import lax
from jax.experimental import pallas as pl
from jax.experimental.pallas import tpu as pltpu
from jax.experimental.pallas import tpu_sc as plsc

N = 10000
D = 128
NC = 2
NS = 16
L = 16
NW = NC * NS
K = 128
NPAD = 10240
RPT = NPAD // NS

_mesh = plsc.VectorSubcoreMesh(core_axis_name="c", subcore_axis_name="s")

_sc_params = pltpu.CompilerParams()
if "needs_layout_passes" in pltpu.CompilerParams.__dataclass_fields__:
    _sc_params = dataclasses.replace(_sc_params, needs_layout_passes=False)


def _hist_call(idx3d, cpt):
    @functools.partial(
        pl.kernel,
        mesh=_mesh,
        out_type=jax.ShapeDtypeStruct((NC, NPAD, D), jnp.float32),
        compiler_params=_sc_params,
        scratch_types=[
            pltpu.VMEM((cpt, 2, K), jnp.int32),
            pltpu.VMEM((NPAD,), jnp.float32),
            pltpu.VMEM((NS, RPT), jnp.float32),
            pltpu.VMEM((L, D), jnp.float32),
            pltpu.VMEM_SHARED((NS, NPAD), jnp.float32),
            pltpu.SemaphoreType.DMA,
        ],
    )
    def hist(idx_hbm, deg_hbm, idxv, histv, redv, bcast, stage_sh, sem):
        cid = lax.axis_index("c")
        sid = lax.axis_index("s")
        wid = cid * NS + sid
        zero = jnp.zeros((L,), jnp.float32)

        @pl.loop(0, NPAD // L)
        def _(i):
            histv[pl.ds(i * L, L)] = zero

        pltpu.async_copy(idx_hbm.at[pl.ds(wid * cpt, cpt)], idxv, sem).wait()

        lane = lax.iota(jnp.int32, L)
        pos = lane.astype(jnp.float32)
        last_lane = lane == (L - 1)
        first_lane = lane == 0
        nxt = jnp.minimum(lane + 1, L - 1)
        prv = jnp.maximum(lane - 1, 0)

        @pl.loop(0, cpt)
        def _(c):
            @pl.loop(0, K // L)
            def _(j):
                s = lax.sort(idxv[c, 0, pl.ds(j * L, L)])
                s_next = s.at[nxt].get(mode="promise_in_bounds")
                s_prev = s.at[prv].get(mode="promise_in_bounds")
                is_last = (s != s_next) | last_lane
                is_first = (s != s_prev) | first_lane
                plsc.addupdate_scatter(histv, [s], -pos, mask=is_first)
                plsc.addupdate_scatter(histv, [s], pos + 1.0, mask=is_last)

        pltpu.sync_copy(histv, stage_sh.at[sid])
        plsc.subcore_barrier()

        pltpu.sync_copy(stage_sh.at[:, pl.ds(sid * RPT, RPT)], redv)

        @pl.loop(0, RPT // L)
        def _(i):
            v = redv[0, pl.ds(i * L, L)]
            for r in range(1, NS):
                v = v + redv[r, pl.ds(i * L, L)]
            histv[pl.ds(i * L, L)] = v

        @pl.loop(0, RPT // L)
        def _(g):
            v = histv[pl.ds(g * L, L)]
            for r in range(L):
                row = zero + v[r]
                for j in range(D // L):
                    bcast[r, pl.ds(j * L, L)] = row
            pltpu.sync_copy(
                bcast, deg_hbm.at[cid, pl.ds(sid * RPT + g * L, L)])

    return hist(idx3d)


def _spmm_call(idx3d, y, cpt):
    @functools.partial(
        pl.kernel,
        mesh=_mesh,
        out_type=jax.ShapeDtypeStruct((NC, NPAD, D), jnp.float32),
        compiler_params=_sc_params,
        scratch_types=[
            pltpu.VMEM((4, 2, K), jnp.int32),
            pltpu.VMEM((2, K, D), jnp.float32),
            pltpu.VMEM_SHARED((NPAD, D), jnp.float32),
            pltpu.SemaphoreType.DMA,
            pltpu.SemaphoreType.DMA,
            pltpu.SemaphoreType.DMA,
            pltpu.SemaphoreType.DMA,
            pltpu.SemaphoreType.DMA,
            pltpu.SemaphoreType.DMA,
        ],
    )
    def spmm(idx_hbm, y_hbm, acc_hbm, idxv, gbuf, acc_sh,
             gs0, gs1, is0, is1, is2, is3):
        cid = lax.axis_index("c")
        sid = lax.axis_index("s")
        wid = cid * NS + sid
        base = wid * cpt
        zero_row = jnp.zeros((L,), jnp.float32)

        @pl.loop(0, K)
        def _(i):
            @pl.loop(0, D // L)
            def _(j):
                gbuf[0, i, pl.ds(j * L, L)] = zero_row

        @pl.loop(0, RPT // K)
        def _(i):
            pltpu.sync_copy(gbuf.at[0], acc_sh.at[pl.ds(sid * RPT + i * K, K)])

        plsc.subcore_barrier()

        gsems = (gs0, gs1)
        isems = (is0, is1, is2, is3)

        def load_idx(c, q):
            return pltpu.async_copy(idx_hbm.at[base + c], idxv.at[q], isems[q])

        def idx_wait(q):
            pltpu.make_async_copy(
                idx_hbm.at[base], idxv.at[q], isems[q]).wait()

        def gather(q, b):
            return pltpu.async_copy(
                y_hbm.at[idxv.at[q, 0]], gbuf.at[b], gsems[b])

        def scatter(q, b):
            pltpu.sync_copy(gbuf.at[b], acc_sh.at[idxv.at[q, 1]], add=True)

        for q in range(4):
            load_idx(q, q)
        gpend = [None, None]
        for b in range(2):
            idx_wait(b)
            gpend[b] = gather(b, b)

        @pl.loop(0, cpt // 4 - 1)
        def _(p):
            c = p * 4
            for k in range(4):
                b = k & 1
                q2 = (k + 2) & 3
                gpend[b].wait()
                scatter(k, b)
                load_idx(c + k + 4, k)
                idx_wait(q2)
                gpend[b] = gather(q2, b)

        for k in range(4):
            b = k & 1
            q2 = (k + 2) & 3
            gpend[b].wait()
            scatter(k, b)
            if k < 2:
                idx_wait(q2)
                gpend[b] = gather(q2, b)

        plsc.subcore_barrier()
        pltpu.sync_copy(
            acc_sh.at[pl.ds(sid * RPT, RPT)],
            acc_hbm.at[cid].at[pl.ds(sid * RPT, RPT)],
        )

    return spmm(idx3d, y)


def _matmul_body(x_ref, wt_ref, y_ref):
    y_ref[...] = jnp.dot(
        x_ref[...], wt_ref[...],
        preferred_element_type=jnp.float32,
        precision=lax.Precision.HIGHEST,
    )


def _matmul_call(x_pad, wt):
    bm = 512
    return pl.pallas_call(
        _matmul_body,
        grid=(NPAD // bm,),
        in_specs=[
            pl.BlockSpec((bm, D), lambda i: (i, 0)),
            pl.BlockSpec((D, D), lambda i: (0, 0)),
        ],
        out_specs=pl.BlockSpec((bm, D), lambda i: (i, 0)),
        out_shape=jax.ShapeDtypeStruct((NPAD, D), jnp.float32),
    )(x_pad, wt)


def _scale_body(xh_ref, d0_ref, d1_ref, y_ref):
    deg = d0_ref[0] + d1_ref[0] + 1.0
    y_ref[...] = lax.rsqrt(deg) * xh_ref[...]


def _scale_call(xhat, degs):
    bm = 1024
    return pl.pallas_call(
        _scale_body,
        grid=(NPAD // bm,),
        in_specs=[
            pl.BlockSpec((bm, D), lambda i: (i, 0)),
            pl.BlockSpec((1, bm, D), lambda i: (0, i, 0)),
            pl.BlockSpec((1, bm, D), lambda i: (1, i, 0)),
        ],
        out_specs=pl.BlockSpec((bm, D), lambda i: (i, 0)),
        out_shape=jax.ShapeDtypeStruct((NPAD, D), jnp.float32),
    )(xhat, degs, degs)


def _finalize_body(a0_ref, a1_ref, y_ref, d0_ref, d1_ref, o_ref):
    deg = d0_ref[0] + d1_ref[0] + 1.0
    s = a0_ref[0] + a1_ref[0] + y_ref[...]
    o_ref[...] = lax.rsqrt(deg) * s


def _finalize_call(accs, y, degs):
    bm = 1000
    return pl.pallas_call(
        _finalize_body,
        grid=(N // bm,),
        in_specs=[
            pl.BlockSpec((1, bm, D), lambda i: (0, i, 0)),
            pl.BlockSpec((1, bm, D), lambda i: (1, i, 0)),
            pl.BlockSpec((bm, D), lambda i: (i, 0)),
            pl.BlockSpec((1, bm, D), lambda i: (0, i, 0)),
            pl.BlockSpec((1, bm, D), lambda i: (1, i, 0)),
        ],
        out_specs=pl.BlockSpec((bm, D), lambda i: (i, 0)),
        out_shape=jax.ShapeDtypeStruct((N, D), jnp.float32),
    )(accs, accs, y, degs, degs)


def kernel(edge_index, X, W):
    E = edge_index.shape[1]
    rows = edge_index[0].astype(jnp.int32)
    cols = edge_index[1].astype(jnp.int32)

    cpt = -(-E // (K * NW))
    cpt = (cpt + 7) // 8 * 8
    e_pad = K * NW * cpt
    npad_e = e_pad - E
    pad_idx = N + jnp.arange(npad_e, dtype=jnp.int32) % (NPAD - N)
    rows2d = jnp.concatenate([rows, pad_idx]).reshape(e_pad // K, K)
    cols2d = jnp.concatenate([cols, pad_idx]).reshape(e_pad // K, K)
    idx3d = jnp.stack([cols2d, rows2d], axis=1)

    x_pad = jnp.pad(X, ((0, NPAD - N), (0, 0)))
    wt = W.T

    xhat = _matmul_call(x_pad, wt)
    degs = _hist_call(idx3d, cpt)
    y = _scale_call(xhat, degs)
    accs = _spmm_call(idx3d, y, cpt)
    return _finalize_call(accs, y, degs)


# speedup vs baseline: 41.5589x; 1.0188x over previous
import dataclasses
import functools

import jax
import jax.numpy as jnp
from jax import lax
from jax.experimental import pallas as pl
from jax.experimental.pallas import tpu as pltpu
from jax.experimental.pallas import tpu_sc as plsc

N = 10000
D = 128
NC = 2
NS = 16
L = 16
NW = NC * NS
K = 128
NPAD = 10240
RPT = NPAD // NS

_mesh = plsc.VectorSubcoreMesh(core_axis_name="c", subcore_axis_name="s")

_sc_params = pltpu.CompilerParams()
if "needs_layout_passes" in pltpu.CompilerParams.__dataclass_fields__:
    _sc_params = dataclasses.replace(_sc_params, needs_layout_passes=False)


def _hist_call(idx3d, cpt):
    @functools.partial(
        pl.kernel,
        mesh=_mesh,
        out_type=jax.ShapeDtypeStruct((NC, NPAD, D), jnp.float32),
        compiler_params=_sc_params,
        scratch_types=[
            pltpu.VMEM((cpt, 2, K), jnp.int32),
            pltpu.VMEM((NPAD,), jnp.float32),
            pltpu.VMEM((NS, RPT), jnp.float32),
            pltpu.VMEM((64, D), jnp.float32),
            pltpu.VMEM_SHARED((NS, NPAD), jnp.float32),
            pltpu.SemaphoreType.DMA,
        ],
    )
    def hist(idx_hbm, deg_hbm, idxv, histv, redv, bcast, stage_sh, sem):
        cid = lax.axis_index("c")
        sid = lax.axis_index("s")
        wid = cid * NS + sid
        zero = jnp.zeros((L,), jnp.float32)
        icp = pltpu.async_copy(idx_hbm.at[pl.ds(wid * cpt, cpt)], idxv, sem)

        @pl.loop(0, NPAD // L)
        def _(i):
            histv[pl.ds(i * L, L)] = zero

        icp.wait()

        lane = lax.iota(jnp.int32, L)
        pos = lane.astype(jnp.float32)
        last_lane = lane == (L - 1)
        first_lane = lane == 0
        nxt = jnp.minimum(lane + 1, L - 1)
        prv = jnp.maximum(lane - 1, 0)

        @pl.loop(0, cpt)
        def _(c):
            for j in range(K // L):
                s = lax.sort(idxv[c, 0, pl.ds(j * L, L)])
                s_next = s.at[nxt].get(mode="promise_in_bounds")
                s_prev = s.at[prv].get(mode="promise_in_bounds")
                is_last = (s != s_next) | last_lane
                is_first = (s != s_prev) | first_lane
                plsc.addupdate_scatter(histv, [s], -pos, mask=is_first)
                plsc.addupdate_scatter(histv, [s], pos + 1.0, mask=is_last)

        pltpu.sync_copy(histv, stage_sh.at[sid])
        plsc.subcore_barrier()

        pltpu.sync_copy(stage_sh.at[:, pl.ds(sid * RPT, RPT)], redv)

        @pl.loop(0, RPT // L)
        def _(i):
            v = redv[0, pl.ds(i * L, L)]
            for r in range(1, NS):
                v = v + redv[r, pl.ds(i * L, L)]
            histv[pl.ds(i * L, L)] = v

        BR = 64
        @pl.loop(0, RPT // BR)
        def _(o):
            for g in range(BR // L):
                v = histv[pl.ds(o * BR + g * L, L)]
                for r in range(L):
                    row = zero + v[r]
                    for j in range(D // L):
                        bcast[g * L + r, pl.ds(j * L, L)] = row
            pltpu.sync_copy(
                bcast, deg_hbm.at[cid, pl.ds(sid * RPT + o * BR, BR)])

    return hist(idx3d)


def _spmm_call(idx3d, y, cpt):
    @functools.partial(
        pl.kernel,
        mesh=_mesh,
        out_type=jax.ShapeDtypeStruct((NC, NPAD, D), jnp.float32),
        compiler_params=_sc_params,
        scratch_types=[
            pltpu.VMEM((4, 2, K), jnp.int32),
            pltpu.VMEM((2, K, D), jnp.float32),
            pltpu.VMEM_SHARED((NPAD, D), jnp.float32),
            pltpu.SemaphoreType.DMA,
            pltpu.SemaphoreType.DMA,
            pltpu.SemaphoreType.DMA,
            pltpu.SemaphoreType.DMA,
            pltpu.SemaphoreType.DMA,
            pltpu.SemaphoreType.DMA,
        ],
    )
    def spmm(idx_hbm, y_hbm, acc_hbm, idxv, gbuf, acc_sh,
             gs0, gs1, is0, is1, is2, is3):
        cid = lax.axis_index("c")
        sid = lax.axis_index("s")
        wid = cid * NS + sid
        base = wid * cpt
        zero_row = jnp.zeros((L,), jnp.float32)

        @pl.loop(0, K)
        def _(i):
            @pl.loop(0, D // L)
            def _(j):
                gbuf[0, i, pl.ds(j * L, L)] = zero_row

        zcps = [
            pltpu.async_copy(
                gbuf.at[0], acc_sh.at[pl.ds(sid * RPT + i * K, K)], is0)
            for i in range(RPT // K)
        ]
        for cp in zcps:
            cp.wait()

        plsc.subcore_barrier()

        gsems = (gs0, gs1)
        isems = (is0, is1, is2, is3)

        def load_idx(c, q):
            return pltpu.async_copy(idx_hbm.at[base + c], idxv.at[q], isems[q])

        def idx_wait(q):
            pltpu.make_async_copy(
                idx_hbm.at[base], idxv.at[q], isems[q]).wait()

        def gather(q, b):
            return pltpu.async_copy(
                y_hbm.at[idxv.at[q, 0]], gbuf.at[b], gsems[b])

        def scatter(q, b):
            pltpu.sync_copy(gbuf.at[b], acc_sh.at[idxv.at[q, 1]], add=True)

        for q in range(4):
            load_idx(q, q)
        gpend = [None, None]
        for b in range(2):
            idx_wait(b)
            gpend[b] = gather(b, b)

        @pl.loop(0, cpt // 4 - 1)
        def _(p):
            c = p * 4
            for k in range(4):
                b = k & 1
                q2 = (k + 2) & 3
                gpend[b].wait()
                scatter(k, b)
                load_idx(c + k + 4, k)
                idx_wait(q2)
                gpend[b] = gather(q2, b)

        for k in range(4):
            b = k & 1
            q2 = (k + 2) & 3
            gpend[b].wait()
            scatter(k, b)
            if k < 2:
                idx_wait(q2)
                gpend[b] = gather(q2, b)

        plsc.subcore_barrier()
        pltpu.sync_copy(
            acc_sh.at[pl.ds(sid * RPT, RPT)],
            acc_hbm.at[cid].at[pl.ds(sid * RPT, RPT)],
        )

    return spmm(idx3d, y)


def _matmul_body(x_ref, wt_ref, y_ref):
    y_ref[...] = jnp.dot(
        x_ref[...], wt_ref[...],
        preferred_element_type=jnp.float32,
        precision=lax.Precision.HIGHEST,
    )


def _matmul_call(x_pad, wt):
    bm = 512
    return pl.pallas_call(
        _matmul_body,
        grid=(NPAD // bm,),
        in_specs=[
            pl.BlockSpec((bm, D), lambda i: (i, 0)),
            pl.BlockSpec((D, D), lambda i: (0, 0)),
        ],
        out_specs=pl.BlockSpec((bm, D), lambda i: (i, 0)),
        out_shape=jax.ShapeDtypeStruct((NPAD, D), jnp.float32),
    )(x_pad, wt)


def _scale_body(xh_ref, d0_ref, d1_ref, y_ref):
    deg = d0_ref[0] + d1_ref[0] + 1.0
    y_ref[...] = lax.rsqrt(deg) * xh_ref[...]


def _scale_call(xhat, degs):
    bm = 1024
    return pl.pallas_call(
        _scale_body,
        grid=(NPAD // bm,),
        in_specs=[
            pl.BlockSpec((bm, D), lambda i: (i, 0)),
            pl.BlockSpec((1, bm, D), lambda i: (0, i, 0)),
            pl.BlockSpec((1, bm, D), lambda i: (1, i, 0)),
        ],
        out_specs=pl.BlockSpec((bm, D), lambda i: (i, 0)),
        out_shape=jax.ShapeDtypeStruct((NPAD, D), jnp.float32),
    )(xhat, degs, degs)


def _finalize_body(a0_ref, a1_ref, y_ref, d0_ref, d1_ref, o_ref):
    deg = d0_ref[0] + d1_ref[0] + 1.0
    s = a0_ref[0] + a1_ref[0] + y_ref[...]
    o_ref[...] = lax.rsqrt(deg) * s


def _finalize_call(accs, y, degs):
    bm = 1000
    return pl.pallas_call(
        _finalize_body,
        grid=(N // bm,),
        in_specs=[
            pl.BlockSpec((1, bm, D), lambda i: (0, i, 0)),
            pl.BlockSpec((1, bm, D), lambda i: (1, i, 0)),
            pl.BlockSpec((bm, D), lambda i: (i, 0)),
            pl.BlockSpec((1, bm, D), lambda i: (0, i, 0)),
            pl.BlockSpec((1, bm, D), lambda i: (1, i, 0)),
        ],
        out_specs=pl.BlockSpec((bm, D), lambda i: (i, 0)),
        out_shape=jax.ShapeDtypeStruct((N, D), jnp.float32),
    )(accs, accs, y, degs, degs)


def kernel(edge_index, X, W):
    E = edge_index.shape[1]
    rows = edge_index[0].astype(jnp.int32)
    cols = edge_index[1].astype(jnp.int32)

    cpt = -(-E // (K * NW))
    cpt = (cpt + 7) // 8 * 8
    e_pad = K * NW * cpt
    npad_e = e_pad - E
    pad_idx = N + jnp.arange(npad_e, dtype=jnp.int32) % (NPAD - N)
    rows2d = jnp.concatenate([rows, pad_idx]).reshape(e_pad // K, K)
    cols2d = jnp.concatenate([cols, pad_idx]).reshape(e_pad // K, K)
    idx3d = jnp.stack([cols2d, rows2d], axis=1)

    x_pad = jnp.pad(X, ((0, NPAD - N), (0, 0)))
    wt = W.T

    xhat = _matmul_call(x_pad, wt)
    degs = _hist_call(idx3d, cpt)
    y = _scale_call(xhat, degs)
    accs = _spmm_call(idx3d, y, cpt)
    return _finalize_call(accs, y, degs)
